# Initial kernel scaffold; baseline (speedup 1.0000x reference)
import jax
import jax.numpy as jnp
from jax.experimental import pallas as pl


def kernel(feature_maps_0, feature_maps_1, feature_maps_2, pixel, batch_index, angle):
    raise NotImplementedError("write your pallas kernel here")


# speedup vs baseline: 1.0867x; 1.0867x over previous
import functools

import jax
import jax.numpy as jnp
from jax import lax
from jax.experimental import pallas as pl
from jax.experimental.pallas import tpu as pltpu
from jax.experimental.pallas import tpu_sc as plsc

_SIZE = 7
_P2 = _SIZE * _SIZE
_STRIDES = (4, 8, 16)
_NC = 2
_NS = 16
_NW = _NC * _NS
_L = 16


def _splat_i32(x):
    return jnp.broadcast_to(jnp.asarray(x, jnp.int32), (_L,))


def _splat_f32(x):
    return jnp.broadcast_to(jnp.asarray(x, jnp.float32), (_L,))


@functools.cache
def _build(dims, n_agents):
    a_per = n_agents // _NW
    csum = []
    off = 0
    for (_, _, c) in dims:
        csum.append(off)
        off += c
    ctot = off
    outd = ctot * _P2

    mesh = plsc.VectorSubcoreMesh(core_axis_name="c", subcore_axis_name="s")

    scratch = [
        pltpu.VMEM((a_per * 2,), jnp.float32),
        pltpu.VMEM((a_per,), jnp.int32),
    ]
    for (_, _, c) in dims:
        scratch.append(pltpu.VMEM((_P2,), jnp.int32))
    for (_, _, c) in dims:
        scratch.append(pltpu.VMEM((_P2, c), jnp.float32))
    scratch += [
        pltpu.VMEM((outd,), jnp.float32),
        pltpu.SemaphoreType.DMA,
    ]

    @functools.partial(
        pl.kernel,
        mesh=mesh,
        out_type=jax.ShapeDtypeStruct((n_agents, outd), jnp.float32),
        scratch_types=scratch,
        compiler_params=pltpu.CompilerParams(needs_layout_passes=False,
                                             use_tc_tiling_on_sc=False),
    )
    def crop(t0, t1, t2, pix_hbm, b_hbm, out_hbm,
             pix_v, b_v, idx0, idx1, idx2, r0, r1, r2, outb, gsem):
        tables = (t0, t1, t2)
        idxs = (idx0, idx1, idx2)
        rows = (r0, r1, r2)

        wid = lax.axis_index("s") * _NC + lax.axis_index("c")
        base = wid * a_per
        pltpu.sync_copy(pix_hbm.at[pl.ds(base * 2, a_per * 2)], pix_v)
        pltpu.sync_copy(b_hbm.at[pl.ds(base, a_per)], b_v)

        iota = lax.iota(jnp.int32, _L)
        c23 = _splat_f32(8388608.0)
        zeros = _splat_i32(0)
        ones = _splat_i32(1)

        def agent_body(i, carry):
            ii = jnp.broadcast_to(i, (_L,)).astype(jnp.int32)
            px = plsc.load_gather(pix_v, [ii * 2])
            py = plsc.load_gather(pix_v, [ii * 2 + ones])
            bb = plsc.load_gather(b_v, [ii])

            for s, (h, w, c) in enumerate(dims):
                inv = _splat_f32(1.0 / _STRIDES[s])
                rx = ((px * inv + c23) - c23).astype(jnp.int32)
                ry = ((py * inv + c23) - c23).astype(jnp.int32)
                bhw = bb * _splat_i32(h * w)
                for k in range(0, _P2, _L):
                    pvec = iota + _splat_i32(k)
                    dxv = lax.div(pvec, _splat_i32(_SIZE)) - _splat_i32(3)
                    dyv = lax.rem(pvec, _splat_i32(_SIZE)) - _splat_i32(3)
                    cx = jnp.minimum(jnp.maximum(rx + dxv, zeros),
                                     _splat_i32(h - 1))
                    cy = jnp.minimum(jnp.maximum(ry + dyv, zeros),
                                     _splat_i32(w - 1))
                    idxv = bhw + cx * _splat_i32(w) + cy
                    plsc.store_scatter(idxs[s], [pvec], idxv,
                                       mask=pvec < _splat_i32(_P2))

            handles = [pltpu.async_copy(tables[s].at[idxs[s]], rows[s], gsem)
                       for s in range(len(dims))]
            for hnd in handles:
                hnd.wait()

            for s, (h, w, c) in enumerate(dims):
                def cc_body(cc, carry2, s=s, c=c):
                    col = iota + cc * _L
                    obase = (col + _splat_i32(csum[s])) * _splat_i32(_P2)
                    for p in range(_P2):
                        v = plsc.load_gather(rows[s], [_splat_i32(p), col])
                        plsc.store_scatter(outb, [obase + _splat_i32(p)], v)
                    return carry2
                lax.fori_loop(0, c // _L, cc_body, 0)

            pltpu.sync_copy(outb, out_hbm.at[base + i])
            return carry

        lax.fori_loop(0, a_per, agent_body, 0)

    return crop


def kernel(feature_maps_0, feature_maps_1, feature_maps_2,
           pixel, batch_index, angle):
    feats = (feature_maps_0, feature_maps_1, feature_maps_2)
    dims = tuple((f.shape[2], f.shape[3], f.shape[1]) for f in feats)
    n_agents = pixel.shape[0]
    ctot = sum(f.shape[1] for f in feats)

    tables = [
        jnp.moveaxis(f, 1, 3).reshape(f.shape[0] * f.shape[2] * f.shape[3],
                                      f.shape[1])
        for f in feats
    ]
    b32 = batch_index.astype(jnp.int32)
    out = _build(dims, n_agents)(*tables, pixel.reshape(-1), b32)
    return out.reshape(n_agents, ctot, _SIZE, _SIZE)
